# Initial kernel scaffold; baseline (speedup 1.0000x reference)
import jax
import jax.numpy as jnp
from jax.experimental import pallas as pl


def kernel(x, W, b):
    raise NotImplementedError("write your pallas kernel here")


# speedup vs baseline: 7.7582x; 7.7582x over previous
import jax
import jax.numpy as jnp
from jax.experimental import pallas as pl

IN_F = 784
OUT_F = 128
NK = 20
NB = 512
BB = 8

_MINT = -2147483648
_M7F = 2147483647


def _decode(k):
    s = jnp.where(k < 0, jnp.bitwise_xor(k, _M7F), k)
    return jax.lax.bitcast_convert_type(s, jnp.float32)


def _body(x_ref, w_ref, b_ref, o_ref):
    xb = x_ref[...]
    w = w_ref[...]
    bias = b_ref[...]

    dot = jax.lax.dot_general(
        xb, w, dimension_numbers=(((1,), (1,)), ((), ())),
        preferred_element_type=jnp.float32)

    temp = xb[:, None, :] * w[None, :, :]

    kf = jnp.float32(NK)

    def step(i, carry):
        p_hi, p_lo = carry
        bit = jax.lax.shift_left(jnp.int32(1), jnp.int32(31) - i)
        c_hi = jnp.bitwise_or(p_hi, bit)
        t_hi = _decode(jnp.bitwise_xor(c_hi, _MINT))[:, :, None]
        cnt_hi = jnp.sum(jnp.where(temp >= t_hi, 1.0, 0.0), axis=-1)
        p_hi = jnp.where(cnt_hi >= kf, c_hi, p_hi)

        c_lo = jnp.bitwise_or(p_lo, bit)
        t_lo = _decode(jnp.bitwise_xor(jnp.bitwise_not(c_lo), _MINT))[:, :, None]
        cnt_lo = jnp.sum(jnp.where(temp <= t_lo, 1.0, 0.0), axis=-1)
        p_lo = jnp.where(cnt_lo >= kf, c_lo, p_lo)
        return p_hi, p_lo

    p0 = jnp.zeros((xb.shape[0], OUT_F), jnp.int32)
    p_hi, p_lo = jax.lax.fori_loop(0, 32, step, (p0, p0))

    t_hi = _decode(jnp.bitwise_xor(p_hi, _MINT))
    t_lo = _decode(jnp.bitwise_xor(p_lo, _MINT))

    sum_hi = jnp.sum(jnp.maximum(temp - t_hi[:, :, None], 0.0), axis=-1)
    sum_lo = jnp.sum(jnp.maximum(-temp - t_lo[:, :, None], 0.0), axis=-1)

    o_ref[...] = dot - (kf * t_hi + sum_hi) + (kf * t_lo + sum_lo) + bias


def kernel(x, W, b):
    b2 = b.reshape(1, OUT_F)
    return pl.pallas_call(
        _body,
        grid=(NB // BB,),
        in_specs=[
            pl.BlockSpec((BB, IN_F), lambda i: (i, 0)),
            pl.BlockSpec((OUT_F, IN_F), lambda i: (0, 0)),
            pl.BlockSpec((1, OUT_F), lambda i: (0, 0)),
        ],
        out_specs=pl.BlockSpec((BB, OUT_F), lambda i: (i, 0)),
        out_shape=jax.ShapeDtypeStruct((NB, OUT_F), jnp.float32),
    )(x, W, b2)
